# baseline (speedup 1.0000x reference)
import functools

import jax
import jax.numpy as jnp
from jax import lax
from jax.experimental import pallas as pl
from jax.experimental.pallas import tpu as pltpu
from jax.experimental.pallas import tpu_sc as plsc

N = 10000
E = 320000
D = 128
DG = 3 * D

NC = 2
NS = 16
NW = NC * NS
C = 128
EPT = 10240
EPAD = EPT * NW
NCHUNK = EPT // C
NPAD = 10240
RPT = NPAD // NS

R = 400
GRID = N // R


def _stage1_body(x_ref, h_ref, wx_ref, wh_ref, b_ref, ggc_ref, whh_ref,
                 bhh_ref, xx_ref, m_ref, gh_ref):
    xx = jnp.dot(x_ref[...], wx_ref[...], preferred_element_type=jnp.float32)
    xx += jnp.dot(h_ref[...], wh_ref[...], preferred_element_type=jnp.float32)
    xx = jnp.maximum(xx + b_ref[...], 0.0)
    xx_ref[...] = xx
    m_ref[...] = jnp.dot(xx, ggc_ref[...], preferred_element_type=jnp.float32)
    gh_ref[...] = jnp.dot(xx, whh_ref[...],
                          preferred_element_type=jnp.float32) + bhh_ref[...]


_stage1 = pl.pallas_call(
    _stage1_body,
    grid=(GRID,),
    in_specs=[
        pl.BlockSpec((R, D), lambda i: (i, 0)),
        pl.BlockSpec((R, D), lambda i: (i, 0)),
        pl.BlockSpec((D, D), lambda i: (0, 0)),
        pl.BlockSpec((D, D), lambda i: (0, 0)),
        pl.BlockSpec((1, D), lambda i: (0, 0)),
        pl.BlockSpec((D, D), lambda i: (0, 0)),
        pl.BlockSpec((D, DG), lambda i: (0, 0)),
        pl.BlockSpec((1, DG), lambda i: (0, 0)),
    ],
    out_specs=[
        pl.BlockSpec((R, D), lambda i: (i, 0)),
        pl.BlockSpec((R, D), lambda i: (i, 0)),
        pl.BlockSpec((R, DG), lambda i: (i, 0)),
    ],
    out_shape=[
        jax.ShapeDtypeStruct((N, D), jnp.float32),
        jax.ShapeDtypeStruct((N, D), jnp.float32),
        jax.ShapeDtypeStruct((N, DG), jnp.float32),
    ],
)


def _sc_body(m_hbm, src_hbm, dst_hbm, zeros_hbm, out0, out1,
             src0, dst0, src1, dst1, rows0, rows1, acc_sh, sem0, sem1):
    c = lax.axis_index("c")
    s = lax.axis_index("s")
    wid = s * NC + c
    r0 = s * RPT
    base = wid * EPT

    pltpu.sync_copy(src_hbm.at[pl.ds(base, C)], src0)
    pltpu.sync_copy(dst_hbm.at[pl.ds(base, C)], dst0)
    pltpu.sync_copy(src_hbm.at[pl.ds(base + C, C)], src1)
    pltpu.sync_copy(dst_hbm.at[pl.ds(base + C, C)], dst1)
    pltpu.async_copy(m_hbm.at[src0], rows0, sem0)
    pltpu.async_copy(m_hbm.at[src1], rows1, sem1)

    pltpu.sync_copy(zeros_hbm.at[pl.ds(r0, RPT)], acc_sh.at[pl.ds(r0, RPT)])
    plsc.subcore_barrier()

    def _wait(buf, sem):
        pltpu.make_async_copy(m_hbm.at[pl.ds(0, C)], buf, sem).wait()

    def pair(i, carry):
        j0 = 2 * i
        _wait(rows0, sem0)
        pltpu.sync_copy(rows0, acc_sh.at[dst0], add=True)

        @pl.when(j0 + 2 < NCHUNK)
        def _():
            pltpu.sync_copy(src_hbm.at[pl.ds(base + (j0 + 2) * C, C)], src0)
            pltpu.sync_copy(dst_hbm.at[pl.ds(base + (j0 + 2) * C, C)], dst0)
            pltpu.async_copy(m_hbm.at[src0], rows0, sem0)

        _wait(rows1, sem1)
        pltpu.sync_copy(rows1, acc_sh.at[dst1], add=True)

        @pl.when(j0 + 3 < NCHUNK)
        def _():
            pltpu.sync_copy(src_hbm.at[pl.ds(base + (j0 + 3) * C, C)], src1)
            pltpu.sync_copy(dst_hbm.at[pl.ds(base + (j0 + 3) * C, C)], dst1)
            pltpu.async_copy(m_hbm.at[src1], rows1, sem1)

        return carry

    lax.fori_loop(0, NCHUNK // 2, pair, 0)
    plsc.subcore_barrier()

    @pl.when(c == 0)
    def _():
        pltpu.sync_copy(acc_sh.at[pl.ds(r0, RPT)], out0.at[pl.ds(r0, RPT)])

    @pl.when(c == 1)
    def _():
        pltpu.sync_copy(acc_sh.at[pl.ds(r0, RPT)], out1.at[pl.ds(r0, RPT)])


_sc_segsum = pl.kernel(
    _sc_body,
    out_type=(
        jax.ShapeDtypeStruct((NPAD, D), jnp.float32),
        jax.ShapeDtypeStruct((NPAD, D), jnp.float32),
    ),
    mesh=plsc.VectorSubcoreMesh(core_axis_name="c", subcore_axis_name="s"),
    scratch_types=[
        pltpu.VMEM((C,), jnp.int32),
        pltpu.VMEM((C,), jnp.int32),
        pltpu.VMEM((C,), jnp.int32),
        pltpu.VMEM((C,), jnp.int32),
        pltpu.VMEM((C, D), jnp.float32),
        pltpu.VMEM((C, D), jnp.float32),
        pltpu.VMEM_SHARED((NPAD, D), jnp.float32),
        pltpu.SemaphoreType.DMA,
        pltpu.SemaphoreType.DMA,
    ],
)


def _stage2_body(p0_ref, p1_ref, xx_ref, gh_ref, wih_ref, bih_ref, out_ref):
    agg = p0_ref[...] + p1_ref[...]
    gi = jnp.dot(agg, wih_ref[...],
                 preferred_element_type=jnp.float32) + bih_ref[...]
    gh = gh_ref[...]
    r = jax.nn.sigmoid(gi[:, :D] + gh[:, :D])
    z = jax.nn.sigmoid(gi[:, D:2 * D] + gh[:, D:2 * D])
    n = jnp.tanh(gi[:, 2 * D:] + r * gh[:, 2 * D:])
    out_ref[...] = (1.0 - z) * n + z * xx_ref[...]


_stage2 = pl.pallas_call(
    _stage2_body,
    grid=(GRID,),
    in_specs=[
        pl.BlockSpec((R, D), lambda i: (i, 0)),
        pl.BlockSpec((R, D), lambda i: (i, 0)),
        pl.BlockSpec((R, D), lambda i: (i, 0)),
        pl.BlockSpec((R, DG), lambda i: (i, 0)),
        pl.BlockSpec((D, DG), lambda i: (0, 0)),
        pl.BlockSpec((1, DG), lambda i: (0, 0)),
    ],
    out_specs=pl.BlockSpec((R, D), lambda i: (i, 0)),
    out_shape=jax.ShapeDtypeStruct((N, D), jnp.float32),
)


def kernel(h, x, pos, edge_index_gate, edge_index_cand,
           fc_w, fc_b, ggc_w, w_ih, w_hh, b_ih, b_hh):
    src = edge_index_gate[0].astype(jnp.int32)
    dst = edge_index_gate[1].astype(jnp.int32)
    pad = EPAD - E
    src_p = jnp.concatenate([src, jnp.zeros((pad,), jnp.int32)])

    dst_p = jnp.concatenate([dst, jnp.full((pad,), N, jnp.int32)])


    wx = fc_w[:, :D].T
    wh = fc_w[:, D:].T
    whh = w_hh.T
    wih = w_ih.T
    b = fc_b.reshape(1, D)
    bhh = b_hh.reshape(1, DG)
    bih = b_ih.reshape(1, DG)

    xx, m, gh = _stage1(x, h, wx, wh, b, ggc_w, whh, bhh)

    zeros = jnp.zeros((NPAD, D), jnp.float32)
    p0, p1 = _sc_segsum(m, src_p, dst_p, zeros)

    return _stage2(p0, p1, xx, gh, wih, bih)


# speedup vs baseline: 1.0066x; 1.0066x over previous
import functools

import jax
import jax.numpy as jnp
from jax import lax
from jax.experimental import pallas as pl
from jax.experimental.pallas import tpu as pltpu
from jax.experimental.pallas import tpu_sc as plsc

N = 10000
E = 320000
D = 128
DG = 3 * D

NC = 2
NS = 16
NW = NC * NS
C = 128
EPT = 10240
EPAD = EPT * NW
NCHUNK = EPT // C
NPAD = 10240
RPT = NPAD // NS

R = 400
GRID = N // R


def _stage1_body(x_ref, h_ref, wx_ref, wh_ref, b_ref, ggc_ref, whh_ref,
                 bhh_ref, xx_ref, m_ref, gh_ref):
    xx = jnp.dot(x_ref[...], wx_ref[...], preferred_element_type=jnp.float32)
    xx += jnp.dot(h_ref[...], wh_ref[...], preferred_element_type=jnp.float32)
    xx = jnp.maximum(xx + b_ref[...], 0.0)
    xx_ref[...] = xx
    m_ref[...] = jnp.dot(xx, ggc_ref[...], preferred_element_type=jnp.float32)
    gh_ref[...] = jnp.dot(xx, whh_ref[...],
                          preferred_element_type=jnp.float32) + bhh_ref[...]


_stage1 = pl.pallas_call(
    _stage1_body,
    grid=(GRID,),
    in_specs=[
        pl.BlockSpec((R, D), lambda i: (i, 0)),
        pl.BlockSpec((R, D), lambda i: (i, 0)),
        pl.BlockSpec((D, D), lambda i: (0, 0)),
        pl.BlockSpec((D, D), lambda i: (0, 0)),
        pl.BlockSpec((1, D), lambda i: (0, 0)),
        pl.BlockSpec((D, D), lambda i: (0, 0)),
        pl.BlockSpec((D, DG), lambda i: (0, 0)),
        pl.BlockSpec((1, DG), lambda i: (0, 0)),
    ],
    out_specs=[
        pl.BlockSpec((R, D), lambda i: (i, 0)),
        pl.BlockSpec((R, D), lambda i: (i, 0)),
        pl.BlockSpec((R, DG), lambda i: (i, 0)),
    ],
    out_shape=[
        jax.ShapeDtypeStruct((N, D), jnp.float32),
        jax.ShapeDtypeStruct((N, D), jnp.float32),
        jax.ShapeDtypeStruct((N, DG), jnp.float32),
    ],
)


def _sc_body(m_hbm, src_hbm, dst_hbm, zeros_hbm, out0, out1,
             src0, dst0, src1, dst1, rows0, rows1, acc_sh, sem0, sem1):
    c = lax.axis_index("c")
    s = lax.axis_index("s")
    wid = s * NC + c
    r0 = s * RPT
    base = wid * EPT

    with jax.named_scope("sc_preamble"):
        pltpu.sync_copy(src_hbm.at[pl.ds(base, C)], src0)
        pltpu.sync_copy(dst_hbm.at[pl.ds(base, C)], dst0)
        pltpu.sync_copy(src_hbm.at[pl.ds(base + C, C)], src1)
        pltpu.sync_copy(dst_hbm.at[pl.ds(base + C, C)], dst1)
        pltpu.async_copy(m_hbm.at[src0], rows0, sem0)
        pltpu.async_copy(m_hbm.at[src1], rows1, sem1)

    with jax.named_scope("sc_zeroinit"):
        pltpu.sync_copy(zeros_hbm.at[pl.ds(r0, RPT)], acc_sh.at[pl.ds(r0, RPT)])
    with jax.named_scope("sc_barrier1"):
        plsc.subcore_barrier()

    def _wait(buf, sem):
        pltpu.make_async_copy(m_hbm.at[pl.ds(0, C)], buf, sem).wait()

    def pair(i, carry):
        j0 = 2 * i
        _wait(rows0, sem0)
        pltpu.sync_copy(rows0, acc_sh.at[dst0], add=True)

        @pl.when(j0 + 2 < NCHUNK)
        def _():
            pltpu.sync_copy(src_hbm.at[pl.ds(base + (j0 + 2) * C, C)], src0)
            pltpu.sync_copy(dst_hbm.at[pl.ds(base + (j0 + 2) * C, C)], dst0)
            pltpu.async_copy(m_hbm.at[src0], rows0, sem0)

        _wait(rows1, sem1)
        pltpu.sync_copy(rows1, acc_sh.at[dst1], add=True)

        @pl.when(j0 + 3 < NCHUNK)
        def _():
            pltpu.sync_copy(src_hbm.at[pl.ds(base + (j0 + 3) * C, C)], src1)
            pltpu.sync_copy(dst_hbm.at[pl.ds(base + (j0 + 3) * C, C)], dst1)
            pltpu.async_copy(m_hbm.at[src1], rows1, sem1)

        return carry

    with jax.named_scope("sc_mainloop"):
        lax.fori_loop(0, NCHUNK // 2, pair, 0)
    with jax.named_scope("sc_barrier2"):
        plsc.subcore_barrier()

    @pl.when(c == 0)
    def _():
        pltpu.sync_copy(acc_sh.at[pl.ds(r0, RPT)], out0.at[pl.ds(r0, RPT)])

    @pl.when(c == 1)
    def _():
        pltpu.sync_copy(acc_sh.at[pl.ds(r0, RPT)], out1.at[pl.ds(r0, RPT)])


_sc_segsum = pl.kernel(
    _sc_body,
    out_type=(
        jax.ShapeDtypeStruct((NPAD, D), jnp.float32),
        jax.ShapeDtypeStruct((NPAD, D), jnp.float32),
    ),
    mesh=plsc.VectorSubcoreMesh(core_axis_name="c", subcore_axis_name="s"),
    scratch_types=[
        pltpu.VMEM((C,), jnp.int32),
        pltpu.VMEM((C,), jnp.int32),
        pltpu.VMEM((C,), jnp.int32),
        pltpu.VMEM((C,), jnp.int32),
        pltpu.VMEM((C, D), jnp.float32),
        pltpu.VMEM((C, D), jnp.float32),
        pltpu.VMEM_SHARED((NPAD, D), jnp.float32),
        pltpu.SemaphoreType.DMA,
        pltpu.SemaphoreType.DMA,
    ],
)


def _stage2_body(p0_ref, p1_ref, xx_ref, gh_ref, wih_ref, bih_ref, out_ref):
    agg = p0_ref[...] + p1_ref[...]
    gi = jnp.dot(agg, wih_ref[...],
                 preferred_element_type=jnp.float32) + bih_ref[...]
    gh = gh_ref[...]
    r = jax.nn.sigmoid(gi[:, :D] + gh[:, :D])
    z = jax.nn.sigmoid(gi[:, D:2 * D] + gh[:, D:2 * D])
    n = jnp.tanh(gi[:, 2 * D:] + r * gh[:, 2 * D:])
    out_ref[...] = (1.0 - z) * n + z * xx_ref[...]


_stage2 = pl.pallas_call(
    _stage2_body,
    grid=(GRID,),
    in_specs=[
        pl.BlockSpec((R, D), lambda i: (i, 0)),
        pl.BlockSpec((R, D), lambda i: (i, 0)),
        pl.BlockSpec((R, D), lambda i: (i, 0)),
        pl.BlockSpec((R, DG), lambda i: (i, 0)),
        pl.BlockSpec((D, DG), lambda i: (0, 0)),
        pl.BlockSpec((1, DG), lambda i: (0, 0)),
    ],
    out_specs=pl.BlockSpec((R, D), lambda i: (i, 0)),
    out_shape=jax.ShapeDtypeStruct((N, D), jnp.float32),
)


def kernel(h, x, pos, edge_index_gate, edge_index_cand,
           fc_w, fc_b, ggc_w, w_ih, w_hh, b_ih, b_hh):
    src = edge_index_gate[0].astype(jnp.int32)
    dst = edge_index_gate[1].astype(jnp.int32)
    pad = EPAD - E
    src_p = jnp.concatenate([src, jnp.zeros((pad,), jnp.int32)])

    dst_p = jnp.concatenate([dst, jnp.full((pad,), N, jnp.int32)])


    wx = fc_w[:, :D].T
    wh = fc_w[:, D:].T
    whh = w_hh.T
    wih = w_ih.T
    b = fc_b.reshape(1, D)
    bhh = b_hh.reshape(1, DG)
    bih = b_ih.reshape(1, DG)

    xx, m, gh = _stage1(x, h, wx, wh, b, ggc_w, whh, bhh)

    zeros = jnp.zeros((NPAD, D), jnp.float32)
    p0, p1 = _sc_segsum(m, src_p, dst_p, zeros)

    return _stage2(p0, p1, xx, gh, wih, bih)


# speedup vs baseline: 1.0088x; 1.0022x over previous
import functools

import jax
import jax.numpy as jnp
from jax import lax
from jax.experimental import pallas as pl
from jax.experimental.pallas import tpu as pltpu
from jax.experimental.pallas import tpu_sc as plsc

N = 10000
E = 320000
D = 128
DG = 3 * D

NC = 2
NS = 16
NW = NC * NS
C = 128
EPT = 10240
EPAD = EPT * NW
NCHUNK = EPT // C
NPAD = 10240
RPT = NPAD // NS

R = 400
GRID = N // R


def _stage1_body(x_ref, h_ref, wx_ref, wh_ref, b_ref, ggc_ref, whh_ref,
                 bhh_ref, xx_ref, m_ref, gh_ref):
    xx = jnp.dot(x_ref[...], wx_ref[...], preferred_element_type=jnp.float32)
    xx += jnp.dot(h_ref[...], wh_ref[...], preferred_element_type=jnp.float32)
    xx = jnp.maximum(xx + b_ref[...], 0.0)
    xx_ref[...] = xx
    m_ref[...] = jnp.dot(xx, ggc_ref[...], preferred_element_type=jnp.float32)
    gh_ref[...] = jnp.dot(xx, whh_ref[...],
                          preferred_element_type=jnp.float32) + bhh_ref[...]


_stage1 = pl.pallas_call(
    _stage1_body,
    grid=(GRID,),
    in_specs=[
        pl.BlockSpec((R, D), lambda i: (i, 0)),
        pl.BlockSpec((R, D), lambda i: (i, 0)),
        pl.BlockSpec((D, D), lambda i: (0, 0)),
        pl.BlockSpec((D, D), lambda i: (0, 0)),
        pl.BlockSpec((1, D), lambda i: (0, 0)),
        pl.BlockSpec((D, D), lambda i: (0, 0)),
        pl.BlockSpec((D, DG), lambda i: (0, 0)),
        pl.BlockSpec((1, DG), lambda i: (0, 0)),
    ],
    out_specs=[
        pl.BlockSpec((R, D), lambda i: (i, 0)),
        pl.BlockSpec((R, D), lambda i: (i, 0)),
        pl.BlockSpec((R, DG), lambda i: (i, 0)),
    ],
    out_shape=[
        jax.ShapeDtypeStruct((N, D), jnp.float32),
        jax.ShapeDtypeStruct((N, D), jnp.float32),
        jax.ShapeDtypeStruct((N, DG), jnp.float32),
    ],
)


def _sc_body(m_hbm, src_hbm, dst_hbm, zeros_hbm, out0, out1,
             src0, dst0, src1, dst1, rows0, rows1, acc_sh, sem0, sem1):
    c = lax.axis_index("c")
    s = lax.axis_index("s")
    wid = s * NC + c
    r0 = s * RPT
    base = wid * EPT

    with jax.named_scope("sc_preamble"):
        pltpu.sync_copy(src_hbm.at[pl.ds(base, C)], src0)
        pltpu.sync_copy(dst_hbm.at[pl.ds(base, C)], dst0)
        pltpu.sync_copy(src_hbm.at[pl.ds(base + C, C)], src1)
        pltpu.sync_copy(dst_hbm.at[pl.ds(base + C, C)], dst1)
        pltpu.async_copy(m_hbm.at[src0], rows0, sem0)
        pltpu.async_copy(m_hbm.at[src1], rows1, sem1)

    with jax.named_scope("sc_zeroinit"):
        pltpu.sync_copy(zeros_hbm.at[pl.ds(r0, RPT)], acc_sh.at[pl.ds(r0, RPT)])
    with jax.named_scope("sc_barrier1"):
        plsc.subcore_barrier()

    def _wait(buf, sem):
        pltpu.make_async_copy(m_hbm.at[pl.ds(0, C)], buf, sem).wait()

    def pair(i, carry):
        j0 = 2 * i
        _wait(rows0, sem0)
        pltpu.sync_copy(rows0, acc_sh.at[dst0], add=True)

        @pl.when(j0 + 2 < NCHUNK)
        def _():
            pltpu.sync_copy(src_hbm.at[pl.ds(base + (j0 + 2) * C, C)], src0)
            pltpu.sync_copy(dst_hbm.at[pl.ds(base + (j0 + 2) * C, C)], dst0)
            pltpu.async_copy(m_hbm.at[src0], rows0, sem0)

        _wait(rows1, sem1)
        pltpu.sync_copy(rows1, acc_sh.at[dst1], add=True)

        @pl.when(j0 + 3 < NCHUNK)
        def _():
            pltpu.sync_copy(src_hbm.at[pl.ds(base + (j0 + 3) * C, C)], src1)
            pltpu.sync_copy(dst_hbm.at[pl.ds(base + (j0 + 3) * C, C)], dst1)
            pltpu.async_copy(m_hbm.at[src1], rows1, sem1)

        return carry

    with jax.named_scope("sc_mainloop"):
        lax.fori_loop(0, NCHUNK // 2, pair, 0)
    with jax.named_scope("sc_barrier2"):
        plsc.subcore_barrier()

    @pl.when(c == 0)
    def _():
        pltpu.sync_copy(acc_sh.at[pl.ds(r0, RPT)], out0.at[pl.ds(r0, RPT)])

    @pl.when(c == 1)
    def _():
        pltpu.sync_copy(acc_sh.at[pl.ds(r0, RPT)], out1.at[pl.ds(r0, RPT)])


_sc_segsum = pl.kernel(
    _sc_body,
    out_type=(
        jax.ShapeDtypeStruct((NPAD, D), jnp.float32),
        jax.ShapeDtypeStruct((NPAD, D), jnp.float32),
    ),
    mesh=plsc.VectorSubcoreMesh(core_axis_name="c", subcore_axis_name="s"),
    scratch_types=[
        pltpu.VMEM((C,), jnp.int32),
        pltpu.VMEM((C,), jnp.int32),
        pltpu.VMEM((C,), jnp.int32),
        pltpu.VMEM((C,), jnp.int32),
        pltpu.VMEM((C, D), jnp.float32),
        pltpu.VMEM((C, D), jnp.float32),
        pltpu.VMEM_SHARED((NPAD, D), jnp.float32),
        pltpu.SemaphoreType.DMA,
        pltpu.SemaphoreType.DMA,
    ],
)


def _stage2_body(p0_ref, p1_ref, xx_ref, gh_ref, wih_ref, bih_ref, out_ref):
    agg = p0_ref[...] + p1_ref[...]
    gi = jnp.dot(agg, wih_ref[...],
                 preferred_element_type=jnp.float32) + bih_ref[...]
    gh = gh_ref[...]
    r = jax.nn.sigmoid(gi[:, :D] + gh[:, :D])
    z = jax.nn.sigmoid(gi[:, D:2 * D] + gh[:, D:2 * D])
    n = jnp.tanh(gi[:, 2 * D:] + r * gh[:, 2 * D:])
    out_ref[...] = (1.0 - z) * n + z * xx_ref[...]


_stage2 = pl.pallas_call(
    _stage2_body,
    grid=(GRID,),
    in_specs=[
        pl.BlockSpec((R, D), lambda i: (i, 0)),
        pl.BlockSpec((R, D), lambda i: (i, 0)),
        pl.BlockSpec((R, D), lambda i: (i, 0)),
        pl.BlockSpec((R, DG), lambda i: (i, 0)),
        pl.BlockSpec((D, DG), lambda i: (0, 0)),
        pl.BlockSpec((1, DG), lambda i: (0, 0)),
    ],
    out_specs=pl.BlockSpec((R, D), lambda i: (i, 0)),
    out_shape=jax.ShapeDtypeStruct((N, D), jnp.float32),
)


def kernel(h, x, pos, edge_index_gate, edge_index_cand,
           fc_w, fc_b, ggc_w, w_ih, w_hh, b_ih, b_hh):
    src = edge_index_gate[0].astype(jnp.int32)
    dst = edge_index_gate[1].astype(jnp.int32)
    pad = EPAD - E
    src_p = jnp.concatenate([src, jnp.zeros((pad,), jnp.int32)])

    pad_dst = N + (jnp.arange(pad, dtype=jnp.int32) % (NPAD - N))
    dst_p = jnp.concatenate([dst, pad_dst])


    wx = fc_w[:, :D].T
    wh = fc_w[:, D:].T
    whh = w_hh.T
    wih = w_ih.T
    b = fc_b.reshape(1, D)
    bhh = b_hh.reshape(1, DG)
    bih = b_ih.reshape(1, DG)

    xx, m, gh = _stage1(x, h, wx, wh, b, ggc_w, whh, bhh)

    zeros = jnp.zeros((NPAD, D), jnp.float32)
    p0, p1 = _sc_segsum(m, src_p, dst_p, zeros)

    return _stage2(p0, p1, xx, gh, wih, bih)


# speedup vs baseline: 2.2859x; 2.2660x over previous
import functools

import jax
import jax.numpy as jnp
from jax import lax
from jax.experimental import pallas as pl
from jax.experimental.pallas import tpu as pltpu
from jax.experimental.pallas import tpu_sc as plsc

N = 10000
E = 320000
D = 128
DG = 3 * D

NC = 2
NS = 16
NW = NC * NS
C = 128
EPT = 10240
EPAD = EPT * NW
NCHUNK = EPT // C
NPAD = 10240
RPT = NPAD // NS

R = 400
GRID = N // R


def _stage1_body(x_ref, h_ref, wx_ref, wh_ref, b_ref, ggc_ref, whh_ref,
                 bhh_ref, xx_ref, m_ref, gh_ref):
    xx = jnp.dot(x_ref[...], wx_ref[...], preferred_element_type=jnp.float32)
    xx += jnp.dot(h_ref[...], wh_ref[...], preferred_element_type=jnp.float32)
    xx = jnp.maximum(xx + b_ref[...], 0.0)
    xx_ref[...] = xx
    m_ref[...] = jnp.dot(xx, ggc_ref[...], preferred_element_type=jnp.float32)
    gh_ref[...] = jnp.dot(xx, whh_ref[...],
                          preferred_element_type=jnp.float32) + bhh_ref[...]


_stage1 = pl.pallas_call(
    _stage1_body,
    grid=(GRID,),
    in_specs=[
        pl.BlockSpec((R, D), lambda i: (i, 0)),
        pl.BlockSpec((R, D), lambda i: (i, 0)),
        pl.BlockSpec((D, D), lambda i: (0, 0)),
        pl.BlockSpec((D, D), lambda i: (0, 0)),
        pl.BlockSpec((1, D), lambda i: (0, 0)),
        pl.BlockSpec((D, D), lambda i: (0, 0)),
        pl.BlockSpec((D, DG), lambda i: (0, 0)),
        pl.BlockSpec((1, DG), lambda i: (0, 0)),
    ],
    out_specs=[
        pl.BlockSpec((R, D), lambda i: (i, 0)),
        pl.BlockSpec((R, D), lambda i: (i, 0)),
        pl.BlockSpec((R, DG), lambda i: (i, 0)),
    ],
    out_shape=[
        jax.ShapeDtypeStruct((N, D), jnp.float32),
        jax.ShapeDtypeStruct((N, D), jnp.float32),
        jax.ShapeDtypeStruct((N, DG), jnp.float32),
    ],
)


def _sc_body(m_hbm, src_hbm, dst_hbm, zeros_hbm, out0, out1,
             src0, dst0, src1, dst1, rows0, rows1, acc_sh, sem0, sem1):
    c = lax.axis_index("c")
    s = lax.axis_index("s")
    wid = s * NC + c
    r0 = s * RPT
    base = wid * EPT

    with jax.named_scope("sc_preamble"):
        pltpu.sync_copy(src_hbm.at[pl.ds(base, C)], src0)
        pltpu.sync_copy(dst_hbm.at[pl.ds(base, C)], dst0)
        pltpu.sync_copy(src_hbm.at[pl.ds(base + C, C)], src1)
        pltpu.sync_copy(dst_hbm.at[pl.ds(base + C, C)], dst1)
        pltpu.async_copy(m_hbm.at[src0], rows0, sem0)
        pltpu.async_copy(m_hbm.at[src1], rows1, sem1)

    with jax.named_scope("sc_zeroinit"):
        pltpu.sync_copy(zeros_hbm.at[pl.ds(r0, RPT)], acc_sh.at[pl.ds(r0, RPT)])
    with jax.named_scope("sc_barrier1"):
        plsc.subcore_barrier()

    def _wait(buf, sem):
        pltpu.make_async_copy(m_hbm.at[pl.ds(0, C)], buf, sem).wait()

    def pair(i, carry):
        j0 = 2 * i
        _wait(rows0, sem0)
        pltpu.sync_copy(rows0, acc_sh.at[dst0], add=True)

        @pl.when(j0 + 2 < NCHUNK)
        def _():
            pltpu.sync_copy(src_hbm.at[pl.ds(base + (j0 + 2) * C, C)], src0)
            pltpu.sync_copy(dst_hbm.at[pl.ds(base + (j0 + 2) * C, C)], dst0)
            pltpu.async_copy(m_hbm.at[src0], rows0, sem0)

        _wait(rows1, sem1)
        pltpu.sync_copy(rows1, acc_sh.at[dst1], add=True)

        @pl.when(j0 + 3 < NCHUNK)
        def _():
            pltpu.sync_copy(src_hbm.at[pl.ds(base + (j0 + 3) * C, C)], src1)
            pltpu.sync_copy(dst_hbm.at[pl.ds(base + (j0 + 3) * C, C)], dst1)
            pltpu.async_copy(m_hbm.at[src1], rows1, sem1)

        return carry

    with jax.named_scope("sc_mainloop"):
        lax.fori_loop(0, NCHUNK // 2, pair, 0)
    with jax.named_scope("sc_barrier2"):
        plsc.subcore_barrier()

    @pl.when(c == 0)
    def _():
        pltpu.sync_copy(acc_sh.at[pl.ds(r0, RPT)], out0.at[pl.ds(r0, RPT)])

    @pl.when(c == 1)
    def _():
        pltpu.sync_copy(acc_sh.at[pl.ds(r0, RPT)], out1.at[pl.ds(r0, RPT)])


_sc_segsum = pl.kernel(
    _sc_body,
    out_type=(
        jax.ShapeDtypeStruct((NPAD, D), jnp.float32),
        jax.ShapeDtypeStruct((NPAD, D), jnp.float32),
    ),
    mesh=plsc.VectorSubcoreMesh(core_axis_name="c", subcore_axis_name="s"),
    scratch_types=[
        pltpu.VMEM((C,), jnp.int32),
        pltpu.VMEM((C,), jnp.int32),
        pltpu.VMEM((C,), jnp.int32),
        pltpu.VMEM((C,), jnp.int32),
        pltpu.VMEM((C, D), jnp.float32),
        pltpu.VMEM((C, D), jnp.float32),
        pltpu.VMEM_SHARED((NPAD, D), jnp.float32),
        pltpu.SemaphoreType.DMA,
        pltpu.SemaphoreType.DMA,
    ],
)


def _stage2_body(p0_ref, p1_ref, xx_ref, gh_ref, wih_ref, bih_ref, out_ref):
    agg = p0_ref[...] + p1_ref[...]
    gi = jnp.dot(agg, wih_ref[...],
                 preferred_element_type=jnp.float32) + bih_ref[...]
    gh = gh_ref[...]
    r = jax.nn.sigmoid(gi[:, :D] + gh[:, :D])
    z = jax.nn.sigmoid(gi[:, D:2 * D] + gh[:, D:2 * D])
    n = jnp.tanh(gi[:, 2 * D:] + r * gh[:, 2 * D:])
    out_ref[...] = (1.0 - z) * n + z * xx_ref[...]


_stage2 = pl.pallas_call(
    _stage2_body,
    grid=(GRID,),
    in_specs=[
        pl.BlockSpec((R, D), lambda i: (i, 0)),
        pl.BlockSpec((R, D), lambda i: (i, 0)),
        pl.BlockSpec((R, D), lambda i: (i, 0)),
        pl.BlockSpec((R, DG), lambda i: (i, 0)),
        pl.BlockSpec((D, DG), lambda i: (0, 0)),
        pl.BlockSpec((1, DG), lambda i: (0, 0)),
    ],
    out_specs=pl.BlockSpec((R, D), lambda i: (i, 0)),
    out_shape=jax.ShapeDtypeStruct((N, D), jnp.float32),
)


def kernel(h, x, pos, edge_index_gate, edge_index_cand,
           fc_w, fc_b, ggc_w, w_ih, w_hh, b_ih, b_hh):
    src = edge_index_gate[0].astype(jnp.int32)
    dst = edge_index_gate[1].astype(jnp.int32)
    pad = EPAD - E
    pad_src = jnp.arange(pad, dtype=jnp.int32) % N
    src_p = jnp.concatenate([src, pad_src])

    pad_dst = N + (jnp.arange(pad, dtype=jnp.int32) % (NPAD - N))
    dst_p = jnp.concatenate([dst, pad_dst])


    wx = fc_w[:, :D].T
    wh = fc_w[:, D:].T
    whh = w_hh.T
    wih = w_ih.T
    b = fc_b.reshape(1, D)
    bhh = b_hh.reshape(1, DG)
    bih = b_ih.reshape(1, DG)

    xx, m, gh = _stage1(x, h, wx, wh, b, ggc_w, whh, bhh)

    zeros = jnp.zeros((NPAD, D), jnp.float32)
    p0, p1 = _sc_segsum(m, src_p, dst_p, zeros)

    return _stage2(p0, p1, xx, gh, wih, bih)


# speedup vs baseline: 2.7812x; 1.2167x over previous
import functools

import jax
import jax.numpy as jnp
from jax import lax
from jax.experimental import pallas as pl
from jax.experimental.pallas import tpu as pltpu
from jax.experimental.pallas import tpu_sc as plsc

N = 10000
E = 320000
D = 128
DG = 3 * D

NC = 2
NS = 16
NW = NC * NS
C = 128
EPT = 10240
EPAD = EPT * NW
NCHUNK = EPT // C
NPAD = 10240
RPT = NPAD // NS

R = 400
GRID = N // R


def _stage1_body(x_ref, h_ref, wx_ref, wh_ref, b_ref, ggc_ref, whh_ref,
                 bhh_ref, xx_ref, m_ref, gh_ref):
    xx = jnp.dot(x_ref[...], wx_ref[...], preferred_element_type=jnp.float32)
    xx += jnp.dot(h_ref[...], wh_ref[...], preferred_element_type=jnp.float32)
    xx = jnp.maximum(xx + b_ref[...], 0.0)
    xx_ref[...] = xx
    m_ref[...] = jnp.dot(xx, ggc_ref[...], preferred_element_type=jnp.float32)
    gh_ref[...] = jnp.dot(xx, whh_ref[...],
                          preferred_element_type=jnp.float32) + bhh_ref[...]


_stage1 = pl.pallas_call(
    _stage1_body,
    grid=(GRID,),
    in_specs=[
        pl.BlockSpec((R, D), lambda i: (i, 0)),
        pl.BlockSpec((R, D), lambda i: (i, 0)),
        pl.BlockSpec((D, D), lambda i: (0, 0)),
        pl.BlockSpec((D, D), lambda i: (0, 0)),
        pl.BlockSpec((1, D), lambda i: (0, 0)),
        pl.BlockSpec((D, D), lambda i: (0, 0)),
        pl.BlockSpec((D, DG), lambda i: (0, 0)),
        pl.BlockSpec((1, DG), lambda i: (0, 0)),
    ],
    out_specs=[
        pl.BlockSpec((R, D), lambda i: (i, 0)),
        pl.BlockSpec((R, D), lambda i: (i, 0)),
        pl.BlockSpec((R, DG), lambda i: (i, 0)),
    ],
    out_shape=[
        jax.ShapeDtypeStruct((N, D), jnp.float32),
        jax.ShapeDtypeStruct((N, D), jnp.float32),
        jax.ShapeDtypeStruct((N, DG), jnp.float32),
    ],
)


def _sc_body(m_hbm, src_hbm, dst_hbm, zeros_hbm, out0, out1,
             sidx, didx, rows0, rows1, gsem0, gsem1, isem0, isem1, acc_sh):
    c = lax.axis_index("c")
    s = lax.axis_index("s")
    wid = s * NC + c
    r0 = s * RPT
    base = wid * EPT

    rows = (rows0, rows1)
    gsem = (gsem0, gsem1)
    isem = (isem0, isem1)
    HALF = NCHUNK // 2

    def _chunk_off(b, k):
        return base + (2 * k + b) * C

    with jax.named_scope("sc_preamble"):
        for b in range(2):
            pltpu.sync_copy(src_hbm.at[pl.ds(_chunk_off(b, 0), C)], sidx[b][0])
            pltpu.sync_copy(dst_hbm.at[pl.ds(_chunk_off(b, 0), C)], didx[b][0])
            pltpu.async_copy(m_hbm.at[sidx[b][0]], rows[b], gsem[b])
            pltpu.async_copy(src_hbm.at[pl.ds(_chunk_off(b, 1), C)],
                             sidx[b][1], isem[b])
            pltpu.async_copy(dst_hbm.at[pl.ds(_chunk_off(b, 1), C)],
                             didx[b][1], isem[b])

    with jax.named_scope("sc_zeroinit"):
        pltpu.sync_copy(zeros_hbm.at[pl.ds(r0, RPT)], acc_sh.at[pl.ds(r0, RPT)])
    with jax.named_scope("sc_barrier1"):
        plsc.subcore_barrier()

    def _gwait(b):
        pltpu.make_async_copy(m_hbm.at[pl.ds(0, C)], rows[b], gsem[b]).wait()

    def _iwait(b, g):
        pltpu.make_async_copy(src_hbm.at[pl.ds(0, C)], sidx[b][g],
                              isem[b]).wait()
        pltpu.make_async_copy(dst_hbm.at[pl.ds(0, C)], didx[b][g],
                              isem[b]).wait()

    def step(t, carry):
        for g in range(2):
            i = 2 * t + g
            for b in range(2):
                _gwait(b)
                pltpu.sync_copy(rows[b], acc_sh.at[didx[b][g]], add=True)

                @pl.when(i + 1 < HALF)
                def _(b=b, g=g):
                    _iwait(b, 1 - g)
                    pltpu.async_copy(m_hbm.at[sidx[b][1 - g]], rows[b],
                                     gsem[b])

                @pl.when(i + 2 < HALF)
                def _(b=b, g=g, i=i):
                    pltpu.async_copy(
                        src_hbm.at[pl.ds(_chunk_off(b, i + 2), C)],
                        sidx[b][g], isem[b])
                    pltpu.async_copy(
                        dst_hbm.at[pl.ds(_chunk_off(b, i + 2), C)],
                        didx[b][g], isem[b])
        return carry

    with jax.named_scope("sc_mainloop"):
        lax.fori_loop(0, HALF // 2, step, 0)
    with jax.named_scope("sc_barrier2"):
        plsc.subcore_barrier()

    @pl.when(c == 0)
    def _():
        pltpu.sync_copy(acc_sh.at[pl.ds(r0, RPT)], out0.at[pl.ds(r0, RPT)])

    @pl.when(c == 1)
    def _():
        pltpu.sync_copy(acc_sh.at[pl.ds(r0, RPT)], out1.at[pl.ds(r0, RPT)])


_sc_segsum = pl.kernel(
    _sc_body,
    out_type=(
        jax.ShapeDtypeStruct((NPAD, D), jnp.float32),
        jax.ShapeDtypeStruct((NPAD, D), jnp.float32),
    ),
    mesh=plsc.VectorSubcoreMesh(core_axis_name="c", subcore_axis_name="s"),
    scratch_types=[
        [[pltpu.VMEM((C,), jnp.int32)] * 2] * 2,
        [[pltpu.VMEM((C,), jnp.int32)] * 2] * 2,
        pltpu.VMEM((C, D), jnp.float32),
        pltpu.VMEM((C, D), jnp.float32),
        pltpu.SemaphoreType.DMA,
        pltpu.SemaphoreType.DMA,
        pltpu.SemaphoreType.DMA,
        pltpu.SemaphoreType.DMA,
        pltpu.VMEM_SHARED((NPAD, D), jnp.float32),
    ],
)


def _stage2_body(p0_ref, p1_ref, xx_ref, gh_ref, wih_ref, bih_ref, out_ref):
    agg = p0_ref[...] + p1_ref[...]
    gi = jnp.dot(agg, wih_ref[...],
                 preferred_element_type=jnp.float32) + bih_ref[...]
    gh = gh_ref[...]
    r = jax.nn.sigmoid(gi[:, :D] + gh[:, :D])
    z = jax.nn.sigmoid(gi[:, D:2 * D] + gh[:, D:2 * D])
    n = jnp.tanh(gi[:, 2 * D:] + r * gh[:, 2 * D:])
    out_ref[...] = (1.0 - z) * n + z * xx_ref[...]


_stage2 = pl.pallas_call(
    _stage2_body,
    grid=(GRID,),
    in_specs=[
        pl.BlockSpec((R, D), lambda i: (i, 0)),
        pl.BlockSpec((R, D), lambda i: (i, 0)),
        pl.BlockSpec((R, D), lambda i: (i, 0)),
        pl.BlockSpec((R, DG), lambda i: (i, 0)),
        pl.BlockSpec((D, DG), lambda i: (0, 0)),
        pl.BlockSpec((1, DG), lambda i: (0, 0)),
    ],
    out_specs=pl.BlockSpec((R, D), lambda i: (i, 0)),
    out_shape=jax.ShapeDtypeStruct((N, D), jnp.float32),
)


def kernel(h, x, pos, edge_index_gate, edge_index_cand,
           fc_w, fc_b, ggc_w, w_ih, w_hh, b_ih, b_hh):
    src = edge_index_gate[0].astype(jnp.int32)
    dst = edge_index_gate[1].astype(jnp.int32)
    pad = EPAD - E
    pad_src = jnp.arange(pad, dtype=jnp.int32) % N
    src_p = jnp.concatenate([src, pad_src])

    pad_dst = N + (jnp.arange(pad, dtype=jnp.int32) % (NPAD - N))
    dst_p = jnp.concatenate([dst, pad_dst])


    wx = fc_w[:, :D].T
    wh = fc_w[:, D:].T
    whh = w_hh.T
    wih = w_ih.T
    b = fc_b.reshape(1, D)
    bhh = b_hh.reshape(1, DG)
    bih = b_ih.reshape(1, DG)

    xx, m, gh = _stage1(x, h, wx, wh, b, ggc_w, whh, bhh)

    zeros = jnp.zeros((NPAD, D), jnp.float32)
    p0, p1 = _sc_segsum(m, src_p, dst_p, zeros)

    return _stage2(p0, p1, xx, gh, wih, bih)


# speedup vs baseline: 2.9214x; 1.0504x over previous
import functools

import jax
import jax.numpy as jnp
from jax import lax
from jax.experimental import pallas as pl
from jax.experimental.pallas import tpu as pltpu
from jax.experimental.pallas import tpu_sc as plsc

N = 10000
E = 320000
D = 128
DG = 3 * D

NC = 2
NS = 16
NW = NC * NS
C = 128
EPT = 10240
EPAD = EPT * NW
NCHUNK = EPT // C
NPAD = 10240
RPT = NPAD // NS

R = 400
GRID = N // R


def _dotT(a, w):
    return lax.dot_general(a, w, (((1,), (1,)), ((), ())),
                           preferred_element_type=jnp.float32)


def _stage1_body(x_ref, h_ref, fcw_ref, b_ref, ggc_ref, xx_ref, m_ref):
    xx = _dotT(x_ref[...], fcw_ref[:, :D]) + _dotT(h_ref[...], fcw_ref[:, D:])
    xx = jnp.maximum(xx + b_ref[...], 0.0)
    xx_ref[...] = xx
    m_ref[...] = jnp.dot(xx, ggc_ref[...], preferred_element_type=jnp.float32)


_stage1 = pl.pallas_call(
    _stage1_body,
    grid=(GRID,),
    in_specs=[
        pl.BlockSpec((R, D), lambda i: (i, 0)),
        pl.BlockSpec((R, D), lambda i: (i, 0)),
        pl.BlockSpec((D, 2 * D), lambda i: (0, 0)),
        pl.BlockSpec((1, D), lambda i: (0, 0)),
        pl.BlockSpec((D, D), lambda i: (0, 0)),
    ],
    out_specs=[
        pl.BlockSpec((R, D), lambda i: (i, 0)),
        pl.BlockSpec((R, D), lambda i: (i, 0)),
    ],
    out_shape=[
        jax.ShapeDtypeStruct((N, D), jnp.float32),
        jax.ShapeDtypeStruct((N, D), jnp.float32),
    ],
)


def _gh_body(xx_ref, whh_ref, bhh_ref, gh_ref):
    gh_ref[...] = _dotT(xx_ref[...], whh_ref[...]) + bhh_ref[...]


_gh = pl.pallas_call(
    _gh_body,
    grid=(GRID,),
    in_specs=[
        pl.BlockSpec((R, D), lambda i: (i, 0)),
        pl.BlockSpec((DG, D), lambda i: (0, 0)),
        pl.BlockSpec((1, DG), lambda i: (0, 0)),
    ],
    out_specs=pl.BlockSpec((R, DG), lambda i: (i, 0)),
    out_shape=jax.ShapeDtypeStruct((N, DG), jnp.float32),
)


def _sc_body(m_hbm, src_hbm, dst_hbm, zeros_hbm, out0, out1,
             sidx, didx, rows0, rows1, gsem0, gsem1, isem0, isem1, acc_sh):
    c = lax.axis_index("c")
    s = lax.axis_index("s")
    wid = s * NC + c
    r0 = s * RPT
    base = wid * EPT

    rows = (rows0, rows1)
    gsem = (gsem0, gsem1)
    isem = (isem0, isem1)
    HALF = NCHUNK // 2

    def _chunk_off(b, k):
        return base + (2 * k + b) * C

    with jax.named_scope("sc_preamble"):
        for b in range(2):
            pltpu.sync_copy(src_hbm.at[pl.ds(_chunk_off(b, 0), C)], sidx[b][0])
            pltpu.sync_copy(dst_hbm.at[pl.ds(_chunk_off(b, 0), C)], didx[b][0])
            pltpu.async_copy(m_hbm.at[sidx[b][0]], rows[b], gsem[b])
            pltpu.async_copy(src_hbm.at[pl.ds(_chunk_off(b, 1), C)],
                             sidx[b][1], isem[b])
            pltpu.async_copy(dst_hbm.at[pl.ds(_chunk_off(b, 1), C)],
                             didx[b][1], isem[b])

    with jax.named_scope("sc_zeroinit"):
        pltpu.sync_copy(zeros_hbm.at[pl.ds(r0, RPT)], acc_sh.at[pl.ds(r0, RPT)])
    with jax.named_scope("sc_barrier1"):
        plsc.subcore_barrier()

    def _gwait(b):
        pltpu.make_async_copy(m_hbm.at[pl.ds(0, C)], rows[b], gsem[b]).wait()

    def _iwait(b, g):
        pltpu.make_async_copy(src_hbm.at[pl.ds(0, C)], sidx[b][g],
                              isem[b]).wait()
        pltpu.make_async_copy(dst_hbm.at[pl.ds(0, C)], didx[b][g],
                              isem[b]).wait()

    def step(t, carry):
        for g in range(2):
            i = 2 * t + g
            for b in range(2):
                _gwait(b)
                pltpu.sync_copy(rows[b], acc_sh.at[didx[b][g]], add=True)

                @pl.when(i + 1 < HALF)
                def _(b=b, g=g):
                    _iwait(b, 1 - g)
                    pltpu.async_copy(m_hbm.at[sidx[b][1 - g]], rows[b],
                                     gsem[b])

                @pl.when(i + 2 < HALF)
                def _(b=b, g=g, i=i):
                    pltpu.async_copy(
                        src_hbm.at[pl.ds(_chunk_off(b, i + 2), C)],
                        sidx[b][g], isem[b])
                    pltpu.async_copy(
                        dst_hbm.at[pl.ds(_chunk_off(b, i + 2), C)],
                        didx[b][g], isem[b])
        return carry

    with jax.named_scope("sc_mainloop"):
        lax.fori_loop(0, HALF // 2, step, 0)
    with jax.named_scope("sc_barrier2"):
        plsc.subcore_barrier()

    @pl.when(c == 0)
    def _():
        pltpu.sync_copy(acc_sh.at[pl.ds(r0, RPT)], out0.at[pl.ds(r0, RPT)])

    @pl.when(c == 1)
    def _():
        pltpu.sync_copy(acc_sh.at[pl.ds(r0, RPT)], out1.at[pl.ds(r0, RPT)])


_sc_segsum = pl.kernel(
    _sc_body,
    out_type=(
        jax.ShapeDtypeStruct((NPAD, D), jnp.float32),
        jax.ShapeDtypeStruct((NPAD, D), jnp.float32),
    ),
    mesh=plsc.VectorSubcoreMesh(core_axis_name="c", subcore_axis_name="s"),
    scratch_types=[
        [[pltpu.VMEM((C,), jnp.int32)] * 2] * 2,
        [[pltpu.VMEM((C,), jnp.int32)] * 2] * 2,
        pltpu.VMEM((C, D), jnp.float32),
        pltpu.VMEM((C, D), jnp.float32),
        pltpu.SemaphoreType.DMA,
        pltpu.SemaphoreType.DMA,
        pltpu.SemaphoreType.DMA,
        pltpu.SemaphoreType.DMA,
        pltpu.VMEM_SHARED((NPAD, D), jnp.float32),
    ],
)


def _stage2_body(p0_ref, p1_ref, xx_ref, gh_ref, wih_ref, bih_ref, out_ref):
    agg = p0_ref[...] + p1_ref[...]
    gi = _dotT(agg, wih_ref[...]) + bih_ref[...]
    gh = gh_ref[...]
    r = jax.nn.sigmoid(gi[:, :D] + gh[:, :D])
    z = jax.nn.sigmoid(gi[:, D:2 * D] + gh[:, D:2 * D])
    n = jnp.tanh(gi[:, 2 * D:] + r * gh[:, 2 * D:])
    out_ref[...] = (1.0 - z) * n + z * xx_ref[...]


_stage2 = pl.pallas_call(
    _stage2_body,
    grid=(GRID,),
    in_specs=[
        pl.BlockSpec((R, D), lambda i: (i, 0)),
        pl.BlockSpec((R, D), lambda i: (i, 0)),
        pl.BlockSpec((R, D), lambda i: (i, 0)),
        pl.BlockSpec((R, DG), lambda i: (i, 0)),
        pl.BlockSpec((DG, D), lambda i: (0, 0)),
        pl.BlockSpec((1, DG), lambda i: (0, 0)),
    ],
    out_specs=pl.BlockSpec((R, D), lambda i: (i, 0)),
    out_shape=jax.ShapeDtypeStruct((N, D), jnp.float32),
)


def kernel(h, x, pos, edge_index_gate, edge_index_cand,
           fc_w, fc_b, ggc_w, w_ih, w_hh, b_ih, b_hh):
    src = edge_index_gate[0].astype(jnp.int32)
    dst = edge_index_gate[1].astype(jnp.int32)
    pad = EPAD - E
    pad_src = jnp.arange(pad, dtype=jnp.int32) % N
    src_p = jnp.concatenate([src, pad_src])

    pad_dst = N + (jnp.arange(pad, dtype=jnp.int32) % (NPAD - N))
    dst_p = jnp.concatenate([dst, pad_dst])


    b = fc_b.reshape(1, D)
    bhh = b_hh.reshape(1, DG)
    bih = b_ih.reshape(1, DG)

    xx, m = _stage1(x, h, fc_w, b, ggc_w)

    zeros = jnp.zeros((NPAD, D), jnp.float32)
    p0, p1 = _sc_segsum(m, src_p, dst_p, zeros)
    gh = _gh(xx, w_hh, bhh)

    return _stage2(p0, p1, xx, gh, w_ih, bih)


# speedup vs baseline: 2.9251x; 1.0013x over previous
import functools

import jax
import jax.numpy as jnp
from jax import lax
from jax.experimental import pallas as pl
from jax.experimental.pallas import tpu as pltpu
from jax.experimental.pallas import tpu_sc as plsc

N = 10000
E = 320000
D = 128
DG = 3 * D

NC = 2
NS = 16
NW = NC * NS
C = 128
TOTAL_CHUNKS = E // C
CH_HI = 80
CH_LO = 76
NEXTRA_W = 17
NPAD = 10240
RPT = NPAD // NS

R = 400
GRID = N // R


def _dotT(a, w):
    return lax.dot_general(a, w, (((1,), (1,)), ((), ())),
                           preferred_element_type=jnp.float32)


def _stage1_body(x_ref, h_ref, fcw_ref, b_ref, ggc_ref, xx_ref, m_ref):
    xx = _dotT(x_ref[...], fcw_ref[:, :D]) + _dotT(h_ref[...], fcw_ref[:, D:])
    xx = jnp.maximum(xx + b_ref[...], 0.0)
    xx_ref[...] = xx
    m_ref[...] = jnp.dot(xx, ggc_ref[...], preferred_element_type=jnp.float32)


_stage1 = pl.pallas_call(
    _stage1_body,
    grid=(GRID,),
    in_specs=[
        pl.BlockSpec((R, D), lambda i: (i, 0)),
        pl.BlockSpec((R, D), lambda i: (i, 0)),
        pl.BlockSpec((D, 2 * D), lambda i: (0, 0)),
        pl.BlockSpec((1, D), lambda i: (0, 0)),
        pl.BlockSpec((D, D), lambda i: (0, 0)),
    ],
    out_specs=[
        pl.BlockSpec((R, D), lambda i: (i, 0)),
        pl.BlockSpec((R, D), lambda i: (i, 0)),
    ],
    out_shape=[
        jax.ShapeDtypeStruct((N, D), jnp.float32),
        jax.ShapeDtypeStruct((N, D), jnp.float32),
    ],
)


def _gh_body(xx_ref, whh_ref, bhh_ref, gh_ref):
    gh_ref[...] = _dotT(xx_ref[...], whh_ref[...]) + bhh_ref[...]


_gh = pl.pallas_call(
    _gh_body,
    grid=(GRID,),
    in_specs=[
        pl.BlockSpec((R, D), lambda i: (i, 0)),
        pl.BlockSpec((DG, D), lambda i: (0, 0)),
        pl.BlockSpec((1, DG), lambda i: (0, 0)),
    ],
    out_specs=pl.BlockSpec((R, DG), lambda i: (i, 0)),
    out_shape=jax.ShapeDtypeStruct((N, DG), jnp.float32),
)


def _sc_body(m_hbm, src_hbm, dst_hbm, zeros_hbm, out0, out1,
             sidx, didx, rows0, rows1, gsem0, gsem1, isem0, isem1, acc_sh):
    c = lax.axis_index("c")
    s = lax.axis_index("s")
    wid = s * NC + c
    r0 = s * RPT
    in_hi = wid < NEXTRA_W
    start = jnp.where(in_hi, wid * CH_HI,
                      NEXTRA_W * CH_HI + (wid - NEXTRA_W) * CH_LO)
    HALF = jnp.where(in_hi, CH_HI // 2, CH_LO // 2)

    rows = (rows0, rows1)
    gsem = (gsem0, gsem1)
    isem = (isem0, isem1)

    def _chunk_off(b, k):
        return (start + 2 * k + b) * C

    with jax.named_scope("sc_preamble"):
        for b in range(2):
            pltpu.sync_copy(src_hbm.at[pl.ds(_chunk_off(b, 0), C)], sidx[b][0])
            pltpu.sync_copy(dst_hbm.at[pl.ds(_chunk_off(b, 0), C)], didx[b][0])
            pltpu.async_copy(m_hbm.at[sidx[b][0]], rows[b], gsem[b])
            pltpu.async_copy(src_hbm.at[pl.ds(_chunk_off(b, 1), C)],
                             sidx[b][1], isem[b])
            pltpu.async_copy(dst_hbm.at[pl.ds(_chunk_off(b, 1), C)],
                             didx[b][1], isem[b])

    with jax.named_scope("sc_zeroinit"):
        pltpu.sync_copy(zeros_hbm.at[pl.ds(r0, RPT)], acc_sh.at[pl.ds(r0, RPT)])
    with jax.named_scope("sc_barrier1"):
        plsc.subcore_barrier()

    def _gwait(b):
        pltpu.make_async_copy(m_hbm.at[pl.ds(0, C)], rows[b], gsem[b]).wait()

    def _iwait(b, g):
        pltpu.make_async_copy(src_hbm.at[pl.ds(0, C)], sidx[b][g],
                              isem[b]).wait()
        pltpu.make_async_copy(dst_hbm.at[pl.ds(0, C)], didx[b][g],
                              isem[b]).wait()

    def step(t, carry):
        for g in range(2):
            i = 2 * t + g
            for b in range(2):
                _gwait(b)
                pltpu.sync_copy(rows[b], acc_sh.at[didx[b][g]], add=True)

                @pl.when(i + 1 < HALF)
                def _(b=b, g=g):
                    _iwait(b, 1 - g)
                    pltpu.async_copy(m_hbm.at[sidx[b][1 - g]], rows[b],
                                     gsem[b])

                @pl.when(i + 2 < HALF)
                def _(b=b, g=g, i=i):
                    pltpu.async_copy(
                        src_hbm.at[pl.ds(_chunk_off(b, i + 2), C)],
                        sidx[b][g], isem[b])
                    pltpu.async_copy(
                        dst_hbm.at[pl.ds(_chunk_off(b, i + 2), C)],
                        didx[b][g], isem[b])
        return carry

    with jax.named_scope("sc_mainloop"):
        lax.fori_loop(0, HALF // 2, step, 0, unroll=False)
    with jax.named_scope("sc_barrier2"):
        plsc.subcore_barrier()

    @pl.when(c == 0)
    def _():
        pltpu.sync_copy(acc_sh.at[pl.ds(r0, RPT)], out0.at[pl.ds(r0, RPT)])

    @pl.when(c == 1)
    def _():
        pltpu.sync_copy(acc_sh.at[pl.ds(r0, RPT)], out1.at[pl.ds(r0, RPT)])


_sc_segsum = pl.kernel(
    _sc_body,
    out_type=(
        jax.ShapeDtypeStruct((NPAD, D), jnp.float32),
        jax.ShapeDtypeStruct((NPAD, D), jnp.float32),
    ),
    mesh=plsc.VectorSubcoreMesh(core_axis_name="c", subcore_axis_name="s"),
    scratch_types=[
        [[pltpu.VMEM((C,), jnp.int32)] * 2] * 2,
        [[pltpu.VMEM((C,), jnp.int32)] * 2] * 2,
        pltpu.VMEM((C, D), jnp.float32),
        pltpu.VMEM((C, D), jnp.float32),
        pltpu.SemaphoreType.DMA,
        pltpu.SemaphoreType.DMA,
        pltpu.SemaphoreType.DMA,
        pltpu.SemaphoreType.DMA,
        pltpu.VMEM_SHARED((NPAD, D), jnp.float32),
    ],
)


def _stage2_body(p0_ref, p1_ref, xx_ref, gh_ref, wih_ref, bih_ref, out_ref):
    agg = p0_ref[...] + p1_ref[...]
    gi = _dotT(agg, wih_ref[...]) + bih_ref[...]
    gh = gh_ref[...]
    r = jax.nn.sigmoid(gi[:, :D] + gh[:, :D])
    z = jax.nn.sigmoid(gi[:, D:2 * D] + gh[:, D:2 * D])
    n = jnp.tanh(gi[:, 2 * D:] + r * gh[:, 2 * D:])
    out_ref[...] = (1.0 - z) * n + z * xx_ref[...]


_stage2 = pl.pallas_call(
    _stage2_body,
    grid=(GRID,),
    in_specs=[
        pl.BlockSpec((R, D), lambda i: (i, 0)),
        pl.BlockSpec((R, D), lambda i: (i, 0)),
        pl.BlockSpec((R, D), lambda i: (i, 0)),
        pl.BlockSpec((R, DG), lambda i: (i, 0)),
        pl.BlockSpec((DG, D), lambda i: (0, 0)),
        pl.BlockSpec((1, DG), lambda i: (0, 0)),
    ],
    out_specs=pl.BlockSpec((R, D), lambda i: (i, 0)),
    out_shape=jax.ShapeDtypeStruct((N, D), jnp.float32),
)


def kernel(h, x, pos, edge_index_gate, edge_index_cand,
           fc_w, fc_b, ggc_w, w_ih, w_hh, b_ih, b_hh):
    src_p = edge_index_gate[0].astype(jnp.int32)
    dst_p = edge_index_gate[1].astype(jnp.int32)


    b = fc_b.reshape(1, D)
    bhh = b_hh.reshape(1, DG)
    bih = b_ih.reshape(1, DG)

    xx, m = _stage1(x, h, fc_w, b, ggc_w)

    zeros = jnp.zeros((NPAD, D), jnp.float32)
    p0, p1 = _sc_segsum(m, src_p, dst_p, zeros)
    gh = _gh(xx, w_hh, bhh)

    return _stage2(p0, p1, xx, gh, w_ih, bih)


# speedup vs baseline: 3.1961x; 1.0926x over previous
import functools

import jax
import jax.numpy as jnp
from jax import lax
from jax.experimental import pallas as pl
from jax.experimental.pallas import tpu as pltpu
from jax.experimental.pallas import tpu_sc as plsc

N = 10000
E = 320000
D = 128
DG = 3 * D

NC = 2
NS = 16
NW = NC * NS
C = 128
TOTAL_CHUNKS = E // C
CH_HI = 80
CH_LO = 76
NEXTRA_W = 17
NPAD = 10240
RPT = NPAD // NS

R = 1000
GRID = N // R


def _dotT(a, w):
    return lax.dot_general(a.astype(jnp.bfloat16), w.astype(jnp.bfloat16),
                           (((1,), (1,)), ((), ())),
                           preferred_element_type=jnp.float32)


def _stage1_body(x_ref, h_ref, fcw_ref, b_ref, ggc_ref, xx_ref, m_ref):
    xx = _dotT(x_ref[...], fcw_ref[:, :D]) + _dotT(h_ref[...], fcw_ref[:, D:])
    xx = jnp.maximum(xx + b_ref[...], 0.0)
    xx_ref[...] = xx
    m_ref[...] = jnp.dot(xx.astype(jnp.bfloat16),
                         ggc_ref[...].astype(jnp.bfloat16),
                         preferred_element_type=jnp.float32)


_stage1 = pl.pallas_call(
    _stage1_body,
    grid=(GRID,),
    in_specs=[
        pl.BlockSpec((R, D), lambda i: (i, 0)),
        pl.BlockSpec((R, D), lambda i: (i, 0)),
        pl.BlockSpec((D, 2 * D), lambda i: (0, 0)),
        pl.BlockSpec((1, D), lambda i: (0, 0)),
        pl.BlockSpec((D, D), lambda i: (0, 0)),
    ],
    out_specs=[
        pl.BlockSpec((R, D), lambda i: (i, 0)),
        pl.BlockSpec((R, D), lambda i: (i, 0)),
    ],
    out_shape=[
        jax.ShapeDtypeStruct((N, D), jnp.float32),
        jax.ShapeDtypeStruct((N, D), jnp.float32),
    ],
)


def _gh_body(xx_ref, whh_ref, bhh_ref, gh_ref):
    gh_ref[...] = _dotT(xx_ref[...], whh_ref[...]) + bhh_ref[...]


_gh = pl.pallas_call(
    _gh_body,
    grid=(GRID,),
    in_specs=[
        pl.BlockSpec((R, D), lambda i: (i, 0)),
        pl.BlockSpec((DG, D), lambda i: (0, 0)),
        pl.BlockSpec((1, DG), lambda i: (0, 0)),
    ],
    out_specs=pl.BlockSpec((R, DG), lambda i: (i, 0)),
    out_shape=jax.ShapeDtypeStruct((N, DG), jnp.float32),
)


def _sc_body(m_hbm, src_hbm, dst_hbm, zeros_hbm, out0, out1,
             sidx, didx, rows0, rows1, gsem0, gsem1, isem0, isem1, acc_sh):
    c = lax.axis_index("c")
    s = lax.axis_index("s")
    wid = s * NC + c
    r0 = s * RPT
    in_hi = wid < NEXTRA_W
    start = jnp.where(in_hi, wid * CH_HI,
                      NEXTRA_W * CH_HI + (wid - NEXTRA_W) * CH_LO)
    HALF = jnp.where(in_hi, CH_HI // 2, CH_LO // 2)

    rows = (rows0, rows1)
    gsem = (gsem0, gsem1)
    isem = (isem0, isem1)

    def _chunk_off(b, k):
        return (start + 2 * k + b) * C

    with jax.named_scope("sc_preamble"):
        for b in range(2):
            pltpu.sync_copy(src_hbm.at[pl.ds(_chunk_off(b, 0), C)], sidx[b][0])
            pltpu.sync_copy(dst_hbm.at[pl.ds(_chunk_off(b, 0), C)], didx[b][0])
            pltpu.async_copy(m_hbm.at[sidx[b][0]], rows[b], gsem[b])
            pltpu.async_copy(src_hbm.at[pl.ds(_chunk_off(b, 1), C)],
                             sidx[b][1], isem[b])
            pltpu.async_copy(dst_hbm.at[pl.ds(_chunk_off(b, 1), C)],
                             didx[b][1], isem[b])

    with jax.named_scope("sc_zeroinit"):
        pltpu.sync_copy(zeros_hbm.at[pl.ds(r0, RPT)], acc_sh.at[pl.ds(r0, RPT)])
    with jax.named_scope("sc_barrier1"):
        plsc.subcore_barrier()

    def _gwait(b):
        pltpu.make_async_copy(m_hbm.at[pl.ds(0, C)], rows[b], gsem[b]).wait()

    def _iwait(b, g):
        pltpu.make_async_copy(src_hbm.at[pl.ds(0, C)], sidx[b][g],
                              isem[b]).wait()
        pltpu.make_async_copy(dst_hbm.at[pl.ds(0, C)], didx[b][g],
                              isem[b]).wait()

    def step(t, carry):
        for g in range(2):
            i = 2 * t + g
            for b in range(2):
                _gwait(b)
                pltpu.sync_copy(rows[b], acc_sh.at[didx[b][g]], add=True)

                @pl.when(i + 1 < HALF)
                def _(b=b, g=g):
                    _iwait(b, 1 - g)
                    pltpu.async_copy(m_hbm.at[sidx[b][1 - g]], rows[b],
                                     gsem[b])

                @pl.when(i + 2 < HALF)
                def _(b=b, g=g, i=i):
                    pltpu.async_copy(
                        src_hbm.at[pl.ds(_chunk_off(b, i + 2), C)],
                        sidx[b][g], isem[b])
                    pltpu.async_copy(
                        dst_hbm.at[pl.ds(_chunk_off(b, i + 2), C)],
                        didx[b][g], isem[b])
        return carry

    with jax.named_scope("sc_mainloop"):
        lax.fori_loop(0, HALF // 2, step, 0, unroll=False)
    with jax.named_scope("sc_barrier2"):
        plsc.subcore_barrier()

    @pl.when(c == 0)
    def _():
        pltpu.sync_copy(acc_sh.at[pl.ds(r0, RPT)], out0.at[pl.ds(r0, RPT)])

    @pl.when(c == 1)
    def _():
        pltpu.sync_copy(acc_sh.at[pl.ds(r0, RPT)], out1.at[pl.ds(r0, RPT)])


_sc_segsum = pl.kernel(
    _sc_body,
    out_type=(
        jax.ShapeDtypeStruct((NPAD, D), jnp.float32),
        jax.ShapeDtypeStruct((NPAD, D), jnp.float32),
    ),
    mesh=plsc.VectorSubcoreMesh(core_axis_name="c", subcore_axis_name="s"),
    scratch_types=[
        [[pltpu.VMEM((C,), jnp.int32)] * 2] * 2,
        [[pltpu.VMEM((C,), jnp.int32)] * 2] * 2,
        pltpu.VMEM((C, D), jnp.float32),
        pltpu.VMEM((C, D), jnp.float32),
        pltpu.SemaphoreType.DMA,
        pltpu.SemaphoreType.DMA,
        pltpu.SemaphoreType.DMA,
        pltpu.SemaphoreType.DMA,
        pltpu.VMEM_SHARED((NPAD, D), jnp.float32),
    ],
)


def _stage2_body(p0_ref, p1_ref, xx_ref, gh_ref, wih_ref, bih_ref, out_ref):
    agg = p0_ref[...] + p1_ref[...]
    gi = _dotT(agg, wih_ref[...]) + bih_ref[...]
    gh = gh_ref[...]
    r = jax.nn.sigmoid(gi[:, :D] + gh[:, :D])
    z = jax.nn.sigmoid(gi[:, D:2 * D] + gh[:, D:2 * D])
    n = jnp.tanh(gi[:, 2 * D:] + r * gh[:, 2 * D:])
    out_ref[...] = (1.0 - z) * n + z * xx_ref[...]


_stage2 = pl.pallas_call(
    _stage2_body,
    grid=(GRID,),
    in_specs=[
        pl.BlockSpec((R, D), lambda i: (i, 0)),
        pl.BlockSpec((R, D), lambda i: (i, 0)),
        pl.BlockSpec((R, D), lambda i: (i, 0)),
        pl.BlockSpec((R, DG), lambda i: (i, 0)),
        pl.BlockSpec((DG, D), lambda i: (0, 0)),
        pl.BlockSpec((1, DG), lambda i: (0, 0)),
    ],
    out_specs=pl.BlockSpec((R, D), lambda i: (i, 0)),
    out_shape=jax.ShapeDtypeStruct((N, D), jnp.float32),
)


def kernel(h, x, pos, edge_index_gate, edge_index_cand,
           fc_w, fc_b, ggc_w, w_ih, w_hh, b_ih, b_hh):
    src_p = edge_index_gate[0].astype(jnp.int32)
    dst_p = edge_index_gate[1].astype(jnp.int32)


    b = fc_b.reshape(1, D)
    bhh = b_hh.reshape(1, DG)
    bih = b_ih.reshape(1, DG)

    xx, m = _stage1(x, h, fc_w, b, ggc_w)

    zeros = jnp.zeros((NPAD, D), jnp.float32)
    p0, p1 = _sc_segsum(m, src_p, dst_p, zeros)
    gh = _gh(xx, w_hh, bhh)

    return _stage2(p0, p1, xx, gh, w_ih, bih)


# speedup vs baseline: 3.2466x; 1.0158x over previous
import functools

import jax
import jax.numpy as jnp
from jax import lax
from jax.experimental import pallas as pl
from jax.experimental.pallas import tpu as pltpu
from jax.experimental.pallas import tpu_sc as plsc

N = 10000
E = 320000
D = 128
DG = 3 * D

NC = 2
NS = 16
NW = NC * NS
C = 128
TOTAL_CHUNKS = E // C
CH_HI = 80
CH_LO = 76
NEXTRA_W = 17
NPAD = 10240
RPT = NPAD // NS

R = 1000
GRID = N // R


def _dotT(a, w):
    return lax.dot_general(a.astype(jnp.bfloat16), w.astype(jnp.bfloat16),
                           (((1,), (1,)), ((), ())),
                           preferred_element_type=jnp.float32)


def _stage1_body(x_ref, h_ref, fcw_ref, b_ref, ggc_ref, xx_ref, m_ref):
    xx = _dotT(x_ref[...], fcw_ref[:, :D]) + _dotT(h_ref[...], fcw_ref[:, D:])
    xx = jnp.maximum(xx + b_ref[...], 0.0)
    xx_ref[...] = xx
    m_ref[...] = jnp.dot(xx.astype(jnp.bfloat16),
                         ggc_ref[...].astype(jnp.bfloat16),
                         preferred_element_type=jnp.float32)


_stage1 = pl.pallas_call(
    _stage1_body,
    grid=(GRID,),
    in_specs=[
        pl.BlockSpec((R, D), lambda i: (i, 0)),
        pl.BlockSpec((R, D), lambda i: (i, 0)),
        pl.BlockSpec((D, 2 * D), lambda i: (0, 0)),
        pl.BlockSpec((1, D), lambda i: (0, 0)),
        pl.BlockSpec((D, D), lambda i: (0, 0)),
    ],
    out_specs=[
        pl.BlockSpec((R, D), lambda i: (i, 0)),
        pl.BlockSpec((R, D), lambda i: (i, 0)),
    ],
    out_shape=[
        jax.ShapeDtypeStruct((N, D), jnp.float32),
        jax.ShapeDtypeStruct((N, D), jnp.float32),
    ],
)


def _gh_body(xx_ref, whh_ref, bhh_ref, gh_ref):
    gh_ref[...] = (_dotT(xx_ref[...], whh_ref[...])
                   + bhh_ref[...]).astype(jnp.bfloat16)


_gh = pl.pallas_call(
    _gh_body,
    grid=(GRID,),
    in_specs=[
        pl.BlockSpec((R, D), lambda i: (i, 0)),
        pl.BlockSpec((DG, D), lambda i: (0, 0)),
        pl.BlockSpec((1, DG), lambda i: (0, 0)),
    ],
    out_specs=pl.BlockSpec((R, DG), lambda i: (i, 0)),
    out_shape=jax.ShapeDtypeStruct((N, DG), jnp.bfloat16),
)


def _sc_body(m_hbm, src_hbm, dst_hbm, zeros_hbm, out0, out1,
             sidx, didx, rows0, rows1, gsem0, gsem1, isem0, isem1, acc_sh):
    c = lax.axis_index("c")
    s = lax.axis_index("s")
    wid = s * NC + c
    r0 = s * RPT
    in_hi = wid < NEXTRA_W
    start = jnp.where(in_hi, wid * CH_HI,
                      NEXTRA_W * CH_HI + (wid - NEXTRA_W) * CH_LO)
    HALF = jnp.where(in_hi, CH_HI // 2, CH_LO // 2)

    rows = (rows0, rows1)
    gsem = (gsem0, gsem1)
    isem = (isem0, isem1)

    def _chunk_off(b, k):
        return (start + 2 * k + b) * C

    with jax.named_scope("sc_preamble"):
        for b in range(2):
            pltpu.sync_copy(src_hbm.at[pl.ds(_chunk_off(b, 0), C)], sidx[b][0])
            pltpu.sync_copy(dst_hbm.at[pl.ds(_chunk_off(b, 0), C)], didx[b][0])
            pltpu.async_copy(m_hbm.at[sidx[b][0]], rows[b], gsem[b])
            pltpu.async_copy(src_hbm.at[pl.ds(_chunk_off(b, 1), C)],
                             sidx[b][1], isem[b])
            pltpu.async_copy(dst_hbm.at[pl.ds(_chunk_off(b, 1), C)],
                             didx[b][1], isem[b])

    with jax.named_scope("sc_zeroinit"):
        pltpu.sync_copy(zeros_hbm.at[pl.ds(r0, RPT)], acc_sh.at[pl.ds(r0, RPT)])
    with jax.named_scope("sc_barrier1"):
        plsc.subcore_barrier()

    def _gwait(b):
        pltpu.make_async_copy(m_hbm.at[pl.ds(0, C)], rows[b], gsem[b]).wait()

    def _iwait(b, g):
        pltpu.make_async_copy(src_hbm.at[pl.ds(0, C)], sidx[b][g],
                              isem[b]).wait()
        pltpu.make_async_copy(dst_hbm.at[pl.ds(0, C)], didx[b][g],
                              isem[b]).wait()

    def step(t, carry):
        for g in range(2):
            i = 2 * t + g
            for b in range(2):
                _gwait(b)
                pltpu.sync_copy(rows[b], acc_sh.at[didx[b][g]], add=True)

                @pl.when(i + 1 < HALF)
                def _(b=b, g=g):
                    _iwait(b, 1 - g)
                    pltpu.async_copy(m_hbm.at[sidx[b][1 - g]], rows[b],
                                     gsem[b])

                @pl.when(i + 2 < HALF)
                def _(b=b, g=g, i=i):
                    pltpu.async_copy(
                        src_hbm.at[pl.ds(_chunk_off(b, i + 2), C)],
                        sidx[b][g], isem[b])
                    pltpu.async_copy(
                        dst_hbm.at[pl.ds(_chunk_off(b, i + 2), C)],
                        didx[b][g], isem[b])
        return carry

    with jax.named_scope("sc_mainloop"):
        lax.fori_loop(0, HALF // 2, step, 0, unroll=False)
    with jax.named_scope("sc_barrier2"):
        plsc.subcore_barrier()

    @pl.when(c == 0)
    def _():
        pltpu.sync_copy(acc_sh.at[pl.ds(r0, RPT)], out0.at[pl.ds(r0, RPT)])

    @pl.when(c == 1)
    def _():
        pltpu.sync_copy(acc_sh.at[pl.ds(r0, RPT)], out1.at[pl.ds(r0, RPT)])


_sc_segsum = pl.kernel(
    _sc_body,
    out_type=(
        jax.ShapeDtypeStruct((NPAD, D), jnp.float32),
        jax.ShapeDtypeStruct((NPAD, D), jnp.float32),
    ),
    mesh=plsc.VectorSubcoreMesh(core_axis_name="c", subcore_axis_name="s"),
    scratch_types=[
        [[pltpu.VMEM((C,), jnp.int32)] * 2] * 2,
        [[pltpu.VMEM((C,), jnp.int32)] * 2] * 2,
        pltpu.VMEM((C, D), jnp.float32),
        pltpu.VMEM((C, D), jnp.float32),
        pltpu.SemaphoreType.DMA,
        pltpu.SemaphoreType.DMA,
        pltpu.SemaphoreType.DMA,
        pltpu.SemaphoreType.DMA,
        pltpu.VMEM_SHARED((NPAD, D), jnp.float32),
    ],
)


def _stage2_body(p0_ref, p1_ref, xx_ref, gh_ref, wih_ref, bih_ref, out_ref):
    agg = p0_ref[...] + p1_ref[...]
    gi = _dotT(agg, wih_ref[...]) + bih_ref[...]
    gh = gh_ref[...].astype(jnp.float32)
    r = jax.nn.sigmoid(gi[:, :D] + gh[:, :D])
    z = jax.nn.sigmoid(gi[:, D:2 * D] + gh[:, D:2 * D])
    n = jnp.tanh(gi[:, 2 * D:] + r * gh[:, 2 * D:])
    out_ref[...] = (1.0 - z) * n + z * xx_ref[...]


_stage2 = pl.pallas_call(
    _stage2_body,
    grid=(GRID,),
    in_specs=[
        pl.BlockSpec((R, D), lambda i: (i, 0)),
        pl.BlockSpec((R, D), lambda i: (i, 0)),
        pl.BlockSpec((R, D), lambda i: (i, 0)),
        pl.BlockSpec((R, DG), lambda i: (i, 0)),
        pl.BlockSpec((DG, D), lambda i: (0, 0)),
        pl.BlockSpec((1, DG), lambda i: (0, 0)),
    ],
    out_specs=pl.BlockSpec((R, D), lambda i: (i, 0)),
    out_shape=jax.ShapeDtypeStruct((N, D), jnp.float32),
)


def kernel(h, x, pos, edge_index_gate, edge_index_cand,
           fc_w, fc_b, ggc_w, w_ih, w_hh, b_ih, b_hh):
    src_p = edge_index_gate[0].astype(jnp.int32)
    dst_p = edge_index_gate[1].astype(jnp.int32)


    b = fc_b.reshape(1, D)
    bhh = b_hh.reshape(1, DG)
    bih = b_ih.reshape(1, DG)

    xx, m = _stage1(x, h, fc_w, b, ggc_w)

    zeros = jnp.zeros((NPAD, D), jnp.float32)
    p0, p1 = _sc_segsum(m, src_p, dst_p, zeros)
    gh = _gh(xx, w_hh, bhh)

    return _stage2(p0, p1, xx, gh, w_ih, bih)
